# baseline (speedup 1.0000x reference)
import jax
import jax.numpy as jnp
from jax import lax
from jax.experimental import pallas as pl
from jax.experimental.pallas import tpu as pltpu, tpu_sc as plsc

_NC, _NS = 2, 16
_NW = _NC * _NS
_ROWS_PER_W = 8192 * 4 // _NW
_CHW = 16384
_PER_W = _ROWS_PER_W * 2048
_NCHUNK = _PER_W // _CHW
_EMB_FLAT = 8192 * 2048


def _sc_body(x_hbm, e_hbm, o_hbm, xb, eb, ob, xsem, esem, osem):
    wid = lax.axis_index("s") * _NC + lax.axis_index("c")
    xbase = wid * _PER_W
    ebase = (wid % 8) * _PER_W

    def x_copy(c, slot):
        return pltpu.make_async_copy(
            x_hbm.at[pl.ds(xbase + c * _CHW, _CHW)], xb.at[slot], xsem.at[slot])

    def e_copy(c, slot):
        return pltpu.make_async_copy(
            e_hbm.at[pl.ds(ebase + c * _CHW, _CHW)], eb.at[slot], esem.at[slot])

    def o_copy(c, slot):
        return pltpu.make_async_copy(
            ob.at[slot], o_hbm.at[pl.ds(xbase + c * _CHW, _CHW)], osem.at[slot])

    x_copy(0, 0).start()
    e_copy(0, 0).start()
    x_copy(1, 1).start()
    e_copy(1, 1).start()

    def step(c, carry):
        slot = lax.rem(c, 2)

        @pl.when(c >= 2)
        def _drain_out():
            o_copy(c - 2, slot).wait()

        x_copy(c, slot).wait()
        e_copy(c, slot).wait()

        def add16(j, carry2):
            ob[slot, pl.ds(j * 16, 16)] = (
                xb[slot, pl.ds(j * 16, 16)] + eb[slot, pl.ds(j * 16, 16)])
            return carry2

        lax.fori_loop(0, _CHW // 16, add16, 0, unroll=8)

        o_copy(c, slot).start()

        @pl.when(c + 2 < _NCHUNK)
        def _refill():
            x_copy(c + 2, slot).start()
            e_copy(c + 2, slot).start()

        return carry

    lax.fori_loop(0, _NCHUNK, step, 0)
    o_copy(_NCHUNK - 2, lax.rem(_NCHUNK - 2, 2)).wait()
    o_copy(_NCHUNK - 1, lax.rem(_NCHUNK - 1, 2)).wait()


def kernel(x, embed):
    B, T, D = x.shape
    n = B * T * D
    x1 = x.reshape(n)
    e1 = embed.reshape(_EMB_FLAT)
    out = pl.kernel(
        _sc_body,
        out_type=jax.ShapeDtypeStruct((n,), jnp.float32),
        mesh=plsc.VectorSubcoreMesh(core_axis_name="c", subcore_axis_name="s"),
        scratch_types=[
            pltpu.VMEM((2, _CHW), jnp.float32),
            pltpu.VMEM((2, _CHW), jnp.float32),
            pltpu.VMEM((2, _CHW), jnp.float32),
            pltpu.SemaphoreType.DMA((2,)),
            pltpu.SemaphoreType.DMA((2,)),
            pltpu.SemaphoreType.DMA((2,)),
        ],
    )(x1, e1)
    return out.reshape(B, T, D)


# speedup vs baseline: 7.9178x; 7.9178x over previous
import jax
import jax.numpy as jnp
from jax.experimental import pallas as pl
from jax.experimental.pallas import tpu as pltpu

_CH = 512
_EB = 1024
_NBUF = 4
_D = 2048


def _chunk_base(s):
    i = s // 8
    r = s % 8
    b = r // 2
    h = r % 2
    return b * 8192 + i * _EB + h * _CH, i, h


def _x_copy(x_hbm, xbuf, xsem, s):
    base, _, _ = _chunk_base(s)
    return pltpu.make_async_copy(
        x_hbm.at[pl.ds(base, _CH), :], xbuf.at[s % _NBUF], xsem.at[s % _NBUF])


def _o_copy(o_hbm, obuf, osem, s):
    base, _, _ = _chunk_base(s)
    return pltpu.make_async_copy(
        obuf.at[s % _NBUF], o_hbm.at[pl.ds(base, _CH), :], osem.at[s % _NBUF])


def _e_copy(e_hbm, ebuf, esem, i):
    return pltpu.make_async_copy(
        e_hbm.at[pl.ds(i * _EB, _EB), :], ebuf.at[i % 2], esem.at[i % 2])


def _body(x_hbm, e_hbm, o_hbm, xbuf, ebuf, obuf, xsem, esem, osem):
    n_steps = 64

    @pl.when(pl.program_id(0) == 0)
    def _prologue():
        for c in range(_NBUF):
            _x_copy(x_hbm, xbuf, xsem, c).start()
        _e_copy(e_hbm, ebuf, esem, 0).start()
        _e_copy(e_hbm, ebuf, esem, 1).start()

    s = pl.program_id(0)
    _, i, h = _chunk_base(s)
    r = s % 8

    @pl.when(r == 0)
    def _embed_turnover():
        @pl.when(jnp.logical_and(i >= 1, i < 7))
        def _prefetch_next():
            _e_copy(e_hbm, ebuf, esem, i + 1).start()

        _e_copy(e_hbm, ebuf, esem, i).wait()

    _x_copy(x_hbm, xbuf, xsem, s).wait()

    @pl.when(s >= _NBUF)
    def _drain_out():
        _o_copy(o_hbm, obuf, osem, s - _NBUF).wait()

    obuf[s % _NBUF] = xbuf[s % _NBUF] + ebuf[i % 2, pl.ds(h * _CH, _CH), :]
    _o_copy(o_hbm, obuf, osem, s).start()

    @pl.when(s < n_steps - _NBUF)
    def _refill_x():
        _x_copy(x_hbm, xbuf, xsem, s + _NBUF).start()

    @pl.when(s == n_steps - 1)
    def _epilogue():
        for k in range(_NBUF):
            _o_copy(o_hbm, obuf, osem, s - (_NBUF - 1) + k).wait()


def kernel(x, embed):
    B, T, D = x.shape
    x2 = x.reshape(B * T, D)
    out = pl.pallas_call(
        _body,
        grid=(64,),
        in_specs=[
            pl.BlockSpec(memory_space=pltpu.HBM),
            pl.BlockSpec(memory_space=pltpu.HBM),
        ],
        out_specs=pl.BlockSpec(memory_space=pltpu.HBM),
        out_shape=jax.ShapeDtypeStruct((B * T, D), x.dtype),
        scratch_shapes=[
            pltpu.VMEM((_NBUF, _CH, _D), jnp.float32),
            pltpu.VMEM((2, _EB, _D), jnp.float32),
            pltpu.VMEM((_NBUF, _CH, _D), jnp.float32),
            pltpu.SemaphoreType.DMA((_NBUF,)),
            pltpu.SemaphoreType.DMA((2,)),
            pltpu.SemaphoreType.DMA((_NBUF,)),
        ],
        compiler_params=pltpu.CompilerParams(
            dimension_semantics=("arbitrary",)),
    )(x2, embed)
    return out.reshape(B, T, D)
